# baseline (speedup 1.0000x reference)
import functools

import jax
import jax.numpy as jnp
from jax import lax
from jax.experimental import pallas as pl
from jax.experimental.pallas import tpu as pltpu
from jax.experimental.pallas import tpu_sc as plsc

_N = 5
_L = 16

_MESH = plsc.VectorSubcoreMesh(core_axis_name="c", subcore_axis_name="s")


@functools.partial(
    pl.kernel,
    out_type=jax.ShapeDtypeStruct((_N, _L), jnp.float32),
    mesh=_MESH,
    scratch_types=[
        pltpu.VMEM((_N, _L), jnp.float32),
        pltpu.VMEM((_N, _L), jnp.float32),
        pltpu.VMEM((_L,), jnp.int32),
    ],
)
def _sc_index_copy(y_hbm, x_hbm, idx_hbm, out_hbm, y_v, out_v, idx_v):
    c = lax.axis_index("c")
    s = lax.axis_index("s")

    @pl.when(jnp.logical_and(c == 0, s == 0))
    def _():
        pltpu.sync_copy(x_hbm, out_v)
        pltpu.sync_copy(y_hbm, y_v)
        pltpu.sync_copy(idx_hbm, idx_v)
        idxvec = idx_v[...]
        y_rows = [y_v[i, :] for i in range(_N)]
        for r in range(_N):
            acc = out_v[r, :]
            for i in range(_N):
                acc = jnp.where(idxvec[i] == r, y_rows[i], acc)
            out_v[r, :] = acc
        pltpu.sync_copy(out_v, out_hbm)


def kernel(y, x, indices):
    y_p = jnp.zeros((_N, _L), jnp.float32).at[:, :_N].set(y)
    x_p = jnp.zeros((_N, _L), jnp.float32).at[:, :_N].set(x)
    idx_p = jnp.zeros((_L,), jnp.int32).at[:_N].set(indices)
    out = _sc_index_copy(y_p, x_p, idx_p)
    return out[:, :_N]


# speedup vs baseline: 1.0943x; 1.0943x over previous
import functools

import jax
import jax.numpy as jnp
from jax import lax
from jax.experimental import pallas as pl
from jax.experimental.pallas import tpu as pltpu
from jax.experimental.pallas import tpu_sc as plsc

_N = 5
_L = 16

_MESH = plsc.VectorSubcoreMesh(core_axis_name="c", subcore_axis_name="s",
                               num_cores=1, num_subcores=1)


@functools.partial(
    pl.kernel,
    out_type=jax.ShapeDtypeStruct((_N, _N), jnp.float32),
    mesh=_MESH,
    scratch_types=[
        pltpu.VMEM((_N, _L), jnp.float32),
        pltpu.VMEM((_N, _L), jnp.float32),
        pltpu.VMEM((_L,), jnp.int32),
        pltpu.SemaphoreType.DMA,
    ],
    compiler_params=pltpu.CompilerParams(use_tc_tiling_on_sc=False),
)
def _sc_index_copy(y_hbm, x_hbm, idx_hbm, out_hbm, y_v, out_v, idx_v, sem):
    c = lax.axis_index("c")
    s = lax.axis_index("s")

    @pl.when(jnp.logical_and(c == 0, s == 0))
    def _():
        cp_x = pltpu.async_copy(x_hbm, out_v.at[:, pl.ds(0, _N)], sem)
        cp_y = pltpu.async_copy(y_hbm, y_v.at[:, pl.ds(0, _N)], sem)
        cp_i = pltpu.async_copy(idx_hbm, idx_v.at[pl.ds(0, _N)], sem)
        cp_x.wait()
        cp_y.wait()
        cp_i.wait()
        idxvec = idx_v[...]
        y_rows = [y_v[i, :] for i in range(_N)]
        for r in range(_N):
            acc = out_v[r, :]
            for i in range(_N):
                acc = jnp.where(idxvec[i] == r, y_rows[i], acc)
            out_v[r, :] = acc
        pltpu.sync_copy(out_v.at[:, pl.ds(0, _N)], out_hbm)


def kernel(y, x, indices):
    return _sc_index_copy(y, x, indices)
